# baseline (speedup 1.0000x reference)
import functools

import jax
import jax.numpy as jnp
from jax.experimental import pallas as pl
from jax.experimental.pallas import tpu as pltpu

_PREC = jax.lax.Precision.DEFAULT
_LANES = 128
_FBLK = 512


def _router_body(e_num, s_len, alpha_ref, x_ref, wr_ref, idx_ref, gate_ref):
    pooled = jnp.sum(x_ref[0], axis=0, keepdims=True) * (1.0 / s_len)
    logits = jnp.dot(pooled, wr_ref[...], precision=_PREC,
                     preferred_element_type=jnp.float32)
    lane = jax.lax.broadcasted_iota(jnp.int32, logits.shape, 1)
    valid = lane < e_num
    l = jnp.where(valid, logits, jnp.float32(-1e30))
    p = jnp.exp(l - jnp.max(l))
    p = jnp.where(valid, p, 0.0)
    p = p / jnp.sum(p)
    big = jnp.int32(1 << 20)
    v1 = jnp.max(p)
    i1 = jnp.min(jnp.where(p >= v1, lane, big))
    p2 = jnp.where(lane == i1, jnp.float32(-1.0), p)
    v2 = jnp.max(p2)
    i2 = jnp.min(jnp.where(p2 >= v2, lane, big))
    a = jnp.clip(alpha_ref[0], 0.1, 1.0)
    scale = (1.0 - a) / (v1 + v2)
    gate_ref[0] = jnp.where(lane == 0, a,
                            jnp.where(lane == 1, scale * v1,
                                      jnp.where(lane == 2, scale * v2, 0.0)))
    idx_row = jnp.where(lane == 0, 0,
                        jnp.where(lane == 1, i1 + 1, i2 + 1))
    idx_ref[0] = idx_row.astype(jnp.int32)


def _dispatch_body(nfb, bidx_ref, gate_ref, x_ref, w1_ref, b1_ref, w2_ref,
                   b2_ref, out_ref):
    b = pl.program_id(0)
    br = pl.program_id(2)
    g = gate_ref[b, br]
    y = None
    for fb in range(nfb):
        lo = fb * _FBLK
        h = jnp.dot(x_ref[0], w1_ref[0, :, lo:lo + _FBLK], precision=_PREC,
                    preferred_element_type=jnp.float32)
        h = jax.nn.gelu(h + b1_ref[0, 0, lo:lo + _FBLK])
        part = jnp.dot(h.astype(jnp.bfloat16), w2_ref[0, lo:lo + _FBLK, :],
                       precision=_PREC, preferred_element_type=jnp.float32)
        y = part if y is None else y + part
    y = g * (y + b2_ref[0, 0])

    @pl.when(br == 0)
    def _init():
        out_ref[0] = y

    @pl.when(br != 0)
    def _acc():
        out_ref[0] += y


def kernel(x, W1, b1, W2, b2, Wr, eW1, eb1, eW2, eb2, alpha):
    B, S, D = x.shape
    F = W1.shape[1]
    E = eW1.shape[0]
    NBR = 3
    NFB = F // _FBLK if F % _FBLK == 0 else 1
    SBLK = 1024 if S % 1024 == 0 else S
    NSB = S // SBLK

    wr_pad = jnp.zeros((D, _LANES), jnp.float32).at[:, :E].set(Wr)
    alpha_arr = jnp.reshape(alpha.astype(jnp.float32), (1,))
    idx_pad, gate_pad = pl.pallas_call(
        functools.partial(_router_body, E, S),
        grid=(B,),
        in_specs=[
            pl.BlockSpec(memory_space=pltpu.SMEM),
            pl.BlockSpec((1, S, D), lambda b: (b, 0, 0)),
            pl.BlockSpec((D, _LANES), lambda b: (0, 0)),
        ],
        out_specs=[
            pl.BlockSpec((1, 1, _LANES), lambda b: (b, 0, 0)),
            pl.BlockSpec((1, 1, _LANES), lambda b: (b, 0, 0)),
        ],
        out_shape=[
            jax.ShapeDtypeStruct((B, 1, _LANES), jnp.int32),
            jax.ShapeDtypeStruct((B, 1, _LANES), jnp.float32),
        ],
        compiler_params=pltpu.CompilerParams(
            vmem_limit_bytes=100 * 1024 * 1024,
        ),
    )(alpha_arr, x, wr_pad)
    bidx = idx_pad[:, 0, :NBR]
    gates = gate_pad[:, 0, :NBR]

    x_bf = x.astype(jnp.bfloat16)
    aW1 = jnp.concatenate([W1[None], eW1], axis=0).astype(jnp.bfloat16)
    aW2 = jnp.concatenate([W2[None], eW2], axis=0).astype(jnp.bfloat16)
    ab1 = jnp.concatenate([b1[None], eb1], axis=0)[:, None, :]
    ab2 = jnp.concatenate([b2[None], eb2], axis=0)[:, None, :]

    grid_spec = pltpu.PrefetchScalarGridSpec(
        num_scalar_prefetch=2,
        grid=(B, NSB, NBR),
        in_specs=[
            pl.BlockSpec((1, SBLK, D),
                         lambda b, sb, br, bidx, gates: (b, sb, 0)),
            pl.BlockSpec((1, D, F),
                         lambda b, sb, br, bidx, gates: (bidx[b, br], 0, 0)),
            pl.BlockSpec((1, 1, F),
                         lambda b, sb, br, bidx, gates: (bidx[b, br], 0, 0)),
            pl.BlockSpec((1, F, D),
                         lambda b, sb, br, bidx, gates: (bidx[b, br], 0, 0)),
            pl.BlockSpec((1, 1, D),
                         lambda b, sb, br, bidx, gates: (bidx[b, br], 0, 0)),
        ],
        out_specs=pl.BlockSpec((1, SBLK, D),
                               lambda b, sb, br, bidx, gates: (b, sb, 0)),
    )
    out = pl.pallas_call(
        functools.partial(_dispatch_body, NFB),
        grid_spec=grid_spec,
        out_shape=jax.ShapeDtypeStruct((B, S, D), jnp.float32),
        compiler_params=pltpu.CompilerParams(
            dimension_semantics=("arbitrary", "arbitrary", "arbitrary"),
            vmem_limit_bytes=100 * 1024 * 1024,
        ),
    )(bidx, gates, x_bf, aW1, ab1, aW2, ab2)
    return out


# speedup vs baseline: 1.1420x; 1.1420x over previous
import functools

import jax
import jax.numpy as jnp
from jax.experimental import pallas as pl
from jax.experimental.pallas import tpu as pltpu

_PREC = jax.lax.Precision.DEFAULT
_LANES = 128


def _router_body(e_num, s_len, alpha_ref, x_ref, wr_ref, idx_ref, gate_ref):
    pooled = jnp.sum(x_ref[0], axis=0, keepdims=True) * (1.0 / s_len)
    logits = jnp.dot(pooled, wr_ref[...], precision=_PREC,
                     preferred_element_type=jnp.float32)
    lane = jax.lax.broadcasted_iota(jnp.int32, logits.shape, 1)
    valid = lane < e_num
    l = jnp.where(valid, logits, jnp.float32(-1e30))
    p = jnp.exp(l - jnp.max(l))
    p = jnp.where(valid, p, 0.0)
    p = p / jnp.sum(p)
    big = jnp.int32(1 << 20)
    v1 = jnp.max(p)
    i1 = jnp.min(jnp.where(p >= v1, lane, big))
    p2 = jnp.where(lane == i1, jnp.float32(-1.0), p)
    v2 = jnp.max(p2)
    i2 = jnp.min(jnp.where(p2 >= v2, lane, big))
    a = jnp.clip(alpha_ref[0], 0.1, 1.0)
    scale = (1.0 - a) / (v1 + v2)
    gate_ref[0] = jnp.where(lane == 0, a,
                            jnp.where(lane == 1, scale * v1,
                                      jnp.where(lane == 2, scale * v2, 0.0)))
    idx_row = jnp.where(lane == 2, i2, i1)
    idx_ref[0] = idx_row.astype(jnp.int32)


def _mlp_branch(x_ref, w1, b1, w2, b2, g, fb, out_ref):
    h = jnp.dot(x_ref[0], w1, precision=_PREC,
                preferred_element_type=jnp.float32)
    h = jax.nn.gelu(h + b1)
    y = jnp.dot(h, w2, precision=_PREC, preferred_element_type=jnp.float32)
    br = pl.program_id(2)

    @pl.when(jnp.logical_and(fb == 0, br == 0))
    def _init():
        out_ref[0] = g * (y + b2)

    @pl.when(jnp.logical_and(fb == 0, br != 0))
    def _acc_bias():
        out_ref[0] += g * (y + b2)

    @pl.when(fb != 0)
    def _acc():
        out_ref[0] += g * y


def _dispatch_body(bidx_ref, gate_ref, x_ref, mw1_ref, mb1_ref, mw2_ref,
                   mb2_ref, ew1_ref, eb1_ref, ew2_ref, eb2_ref, out_ref):
    b = pl.program_id(0)
    fb = pl.program_id(1)
    br = pl.program_id(2)
    g = gate_ref[b, br]

    @pl.when(br == 0)
    def _main():
        _mlp_branch(x_ref, mw1_ref[...], mb1_ref[0], mw2_ref[...],
                    mb2_ref[0], g, fb, out_ref)

    @pl.when(br != 0)
    def _expert():
        _mlp_branch(x_ref, ew1_ref[0], eb1_ref[0, 0], ew2_ref[0],
                    eb2_ref[0, 0], g, fb, out_ref)


def kernel(x, W1, b1, W2, b2, Wr, eW1, eb1, eW2, eb2, alpha):
    B, S, D = x.shape
    F = W1.shape[1]
    E = eW1.shape[0]
    NBR = 3

    FBLK = 512 if F % 512 == 0 else F
    NFB = F // FBLK

    wr_pad = jnp.zeros((D, _LANES), jnp.float32).at[:, :E].set(Wr)
    alpha_arr = jnp.reshape(alpha.astype(jnp.float32), (1,))
    idx_pad, gate_pad = pl.pallas_call(
        functools.partial(_router_body, E, S),
        grid=(B,),
        in_specs=[
            pl.BlockSpec(memory_space=pltpu.SMEM),
            pl.BlockSpec((1, S, D), lambda b: (b, 0, 0)),
            pl.BlockSpec((D, _LANES), lambda b: (0, 0)),
        ],
        out_specs=[
            pl.BlockSpec((1, 1, _LANES), lambda b: (b, 0, 0)),
            pl.BlockSpec((1, 1, _LANES), lambda b: (b, 0, 0)),
        ],
        out_shape=[
            jax.ShapeDtypeStruct((B, 1, _LANES), jnp.int32),
            jax.ShapeDtypeStruct((B, 1, _LANES), jnp.float32),
        ],
        compiler_params=pltpu.CompilerParams(
            vmem_limit_bytes=100 * 1024 * 1024,
        ),
    )(alpha_arr, x, wr_pad)
    bidx = idx_pad[:, 0, :NBR]
    gates = gate_pad[:, 0, :NBR]

    grid_spec = pltpu.PrefetchScalarGridSpec(
        num_scalar_prefetch=2,
        grid=(B, NFB, NBR),
        in_specs=[
            pl.BlockSpec((1, S, D), lambda b, fb, br, bidx, gates: (b, 0, 0)),
            pl.BlockSpec((D, FBLK), lambda b, fb, br, bidx, gates: (0, fb)),
            pl.BlockSpec((1, FBLK), lambda b, fb, br, bidx, gates: (0, fb)),
            pl.BlockSpec((FBLK, D), lambda b, fb, br, bidx, gates: (fb, 0)),
            pl.BlockSpec((1, D), lambda b, fb, br, bidx, gates: (0, 0)),
            pl.BlockSpec((1, D, FBLK),
                         lambda b, fb, br, bidx, gates: (bidx[b, br], 0, fb)),
            pl.BlockSpec((1, 1, FBLK),
                         lambda b, fb, br, bidx, gates: (bidx[b, br], 0, fb)),
            pl.BlockSpec((1, FBLK, D),
                         lambda b, fb, br, bidx, gates: (bidx[b, br], fb, 0)),
            pl.BlockSpec((1, 1, D),
                         lambda b, fb, br, bidx, gates: (bidx[b, br], 0, 0)),
        ],
        out_specs=pl.BlockSpec((1, S, D),
                               lambda b, fb, br, bidx, gates: (b, 0, 0)),
    )
    out = pl.pallas_call(
        _dispatch_body,
        grid_spec=grid_spec,
        out_shape=jax.ShapeDtypeStruct((B, S, D), jnp.float32),
        compiler_params=pltpu.CompilerParams(
            dimension_semantics=("arbitrary", "arbitrary", "arbitrary"),
            vmem_limit_bytes=100 * 1024 * 1024,
        ),
    )(bidx, gates, x, W1, b1[None, :], W2, b2[None, :],
      eW1, eb1[:, None, :], eW2, eb2[:, None, :])
    return out


# speedup vs baseline: 1.1437x; 1.0015x over previous
import functools

import jax
import jax.numpy as jnp
from jax.experimental import pallas as pl
from jax.experimental.pallas import tpu as pltpu

_PREC = jax.lax.Precision.DEFAULT
_LANES = 128


def _router_body(e_num, s_len, alpha_ref, x_ref, wr_ref, idx_ref, gate_ref):
    pooled = jnp.sum(x_ref[0], axis=0, keepdims=True) * (1.0 / s_len)
    logits = jnp.dot(pooled, wr_ref[...], precision=_PREC,
                     preferred_element_type=jnp.float32)
    lane = jax.lax.broadcasted_iota(jnp.int32, logits.shape, 1)
    valid = lane < e_num
    l = jnp.where(valid, logits, jnp.float32(-1e30))
    p = jnp.exp(l - jnp.max(l))
    p = jnp.where(valid, p, 0.0)
    p = p / jnp.sum(p)
    big = jnp.int32(1 << 20)
    v1 = jnp.max(p)
    i1 = jnp.min(jnp.where(p >= v1, lane, big))
    p2 = jnp.where(lane == i1, jnp.float32(-1.0), p)
    v2 = jnp.max(p2)
    i2 = jnp.min(jnp.where(p2 >= v2, lane, big))
    a = jnp.clip(alpha_ref[0], 0.1, 1.0)
    scale = (1.0 - a) / (v1 + v2)
    gate_ref[0] = jnp.where(lane == 0, a,
                            jnp.where(lane == 1, scale * v1,
                                      jnp.where(lane == 2, scale * v2, 0.0)))
    idx_row = jnp.where(lane == 2, i2, i1)
    idx_ref[0] = idx_row.astype(jnp.int32)


def _mlp_branch(x_ref, w1, b1, w2, b2, g, fb, out_ref):
    h = jnp.dot(x_ref[0], w1, precision=_PREC,
                preferred_element_type=jnp.float32)
    h = jax.nn.gelu(h + b1)
    y = jnp.dot(h, w2, precision=_PREC, preferred_element_type=jnp.float32)
    br = pl.program_id(2)

    @pl.when(jnp.logical_and(fb == 0, br == 0))
    def _init():
        out_ref[0] = g * (y + b2)

    @pl.when(jnp.logical_and(fb == 0, br != 0))
    def _acc_bias():
        out_ref[0] += g * (y + b2)

    @pl.when(fb != 0)
    def _acc():
        out_ref[0] += g * y


def _dispatch_body(bidx_ref, gate_ref, x_ref, mw1_ref, mb1_ref, mw2_ref,
                   mb2_ref, ew1_ref, eb1_ref, ew2_ref, eb2_ref, out_ref):
    b = pl.program_id(0)
    fb = pl.program_id(1)
    br = pl.program_id(2)
    g = gate_ref[b, br]

    @pl.when(br == 0)
    def _main():
        _mlp_branch(x_ref, mw1_ref[...], mb1_ref[0], mw2_ref[...],
                    mb2_ref[0], g, fb, out_ref)

    @pl.when(br != 0)
    def _expert():
        _mlp_branch(x_ref, ew1_ref[0], eb1_ref[0, 0], ew2_ref[0],
                    eb2_ref[0, 0], g, fb, out_ref)


def kernel(x, W1, b1, W2, b2, Wr, eW1, eb1, eW2, eb2, alpha):
    B, S, D = x.shape
    F = W1.shape[1]
    E = eW1.shape[0]
    NBR = 3

    FBLK = 512 if F % 512 == 0 else F
    NFB = F // FBLK

    wr_pad = jnp.zeros((D, _LANES), jnp.float32).at[:, :E].set(Wr)
    alpha_arr = jnp.reshape(alpha.astype(jnp.float32), (1,))
    idx_pad, gate_pad = pl.pallas_call(
        functools.partial(_router_body, E, S),
        grid=(B,),
        in_specs=[
            pl.BlockSpec(memory_space=pltpu.SMEM),
            pl.BlockSpec((1, S, D), lambda b: (b, 0, 0)),
            pl.BlockSpec((D, _LANES), lambda b: (0, 0)),
        ],
        out_specs=[
            pl.BlockSpec((1, 1, _LANES), lambda b: (b, 0, 0)),
            pl.BlockSpec((1, 1, _LANES), lambda b: (b, 0, 0)),
        ],
        out_shape=[
            jax.ShapeDtypeStruct((B, 1, _LANES), jnp.int32),
            jax.ShapeDtypeStruct((B, 1, _LANES), jnp.float32),
        ],
        compiler_params=pltpu.CompilerParams(
            vmem_limit_bytes=100 * 1024 * 1024,
        ),
    )(alpha_arr, x, wr_pad)
    bidx = idx_pad[:, 0, :NBR]
    gates = gate_pad[:, 0, :NBR]

    grid_spec = pltpu.PrefetchScalarGridSpec(
        num_scalar_prefetch=2,
        grid=(B, NFB, NBR),
        in_specs=[
            pl.BlockSpec((1, S, D), lambda b, fb, br, bidx, gates: (b, 0, 0)),
            pl.BlockSpec((D, FBLK), lambda b, fb, br, bidx, gates: (0, fb)),
            pl.BlockSpec((1, FBLK), lambda b, fb, br, bidx, gates: (0, fb)),
            pl.BlockSpec((FBLK, D), lambda b, fb, br, bidx, gates: (fb, 0)),
            pl.BlockSpec((1, D), lambda b, fb, br, bidx, gates: (0, 0)),
            pl.BlockSpec((1, D, FBLK),
                         lambda b, fb, br, bidx, gates: (bidx[b, br], 0, fb)),
            pl.BlockSpec((1, 1, FBLK),
                         lambda b, fb, br, bidx, gates: (bidx[b, br], 0, fb)),
            pl.BlockSpec((1, FBLK, D),
                         lambda b, fb, br, bidx, gates: (bidx[b, br], fb, 0)),
            pl.BlockSpec((1, 1, D),
                         lambda b, fb, br, bidx, gates: (bidx[b, br], 0, 0)),
        ],
        out_specs=pl.BlockSpec((1, S, D),
                               lambda b, fb, br, bidx, gates: (b, 0, 0)),
    )
    out = pl.pallas_call(
        _dispatch_body,
        grid_spec=grid_spec,
        out_shape=jax.ShapeDtypeStruct((B, S, D), jnp.float32),
        compiler_params=pltpu.CompilerParams(
            dimension_semantics=("parallel", "arbitrary", "arbitrary"),
            vmem_limit_bytes=100 * 1024 * 1024,
        ),
    )(bidx, gates, x, W1, b1[None, :], W2, b2[None, :],
      eW1, eb1[:, None, :], eW2, eb2[:, None, :])
    return out
